# Initial kernel scaffold; baseline (speedup 1.0000x reference)
import jax
import jax.numpy as jnp
from jax.experimental import pallas as pl


def kernel(x, m):
    raise NotImplementedError("write your pallas kernel here")


# speedup vs baseline: 1.2220x; 1.2220x over previous
import jax
import jax.numpy as jnp
from jax import lax
from jax.experimental import pallas as pl
from jax.experimental.pallas import tpu as pltpu

_N = 16384
_H = 256
_K = 1024
_RATE = 0.999
_BLK = 512
_NBLK = _N // _BLK


def _rownorm(a, eps=1e-12):
    nrm = jnp.sqrt(jnp.sum(a * a, axis=1, keepdims=True))
    return a / jnp.maximum(nrm, eps)


def _stats_body(x_ref, m_ref, esumT_ref, cntT_ref, mn_ref):
    i = pl.program_id(0)

    @pl.when(i == 0)
    def _init():
        mn_ref[...] = _rownorm(m_ref[...])
        esumT_ref[...] = jnp.zeros_like(esumT_ref)
        cntT_ref[...] = jnp.zeros_like(cntT_ref)

    x = x_ref[...]
    xn = _rownorm(x)
    score = lax.dot_general(xn, mn_ref[...], (((1,), (1,)), ((), ())),
                            preferred_element_type=jnp.float32)
    mx = jnp.max(score, axis=1, keepdims=True)
    iota = lax.broadcasted_iota(jnp.int32, score.shape, 1)
    idx = jnp.min(jnp.where(score == mx, iota, _K), axis=1, keepdims=True)
    onehot = (iota == idx).astype(jnp.float32)
    esumT_ref[...] += lax.dot_general(onehot, x, (((0,), (0,)), ((), ())),
                                      preferred_element_type=jnp.float32)
    cntT_ref[...] += lax.dot_general(onehot, jnp.ones((x.shape[0], 8), jnp.float32),
                                     (((0,), (0,)), ((), ())),
                                     preferred_element_type=jnp.float32)


def _out_body(x_ref, m_ref, esumT_ref, cntT_ref, score2_ref, out_ref,
              nd_ref, mn2_ref):
    i = pl.program_id(0)

    @pl.when(i == 0)
    def _init():
        cnt = cntT_ref[...][:, 0:1]
        emeanT = esumT_ref[...] / (cnt + 1e-6)
        nd = m_ref[...] * _RATE + emeanT * (1.0 - _RATE)
        nd_ref[...] = nd
        mn2_ref[...] = _rownorm(nd)

    x = x_ref[...]
    xn = _rownorm(x)
    s2 = lax.dot_general(xn, mn2_ref[...], (((1,), (1,)), ((), ())),
                         preferred_element_type=jnp.float32)
    score2_ref[...] = s2
    mx = jnp.max(s2, axis=1, keepdims=True)
    e = jnp.exp(s2 - mx)
    soft = e / jnp.sum(e, axis=1, keepdims=True)
    out_ref[...] = lax.dot_general(soft, nd_ref[...], (((1,), (0,)), ((), ())),
                                   preferred_element_type=jnp.float32)


def kernel(x, m):
    esumT, cntT = pl.pallas_call(
        _stats_body,
        grid=(_NBLK,),
        in_specs=[pl.BlockSpec((_BLK, _H), lambda i: (i, 0)),
                  pl.BlockSpec((_K, _H), lambda i: (0, 0))],
        out_specs=[pl.BlockSpec((_K, _H), lambda i: (0, 0)),
                   pl.BlockSpec((_K, 8), lambda i: (0, 0))],
        out_shape=[jax.ShapeDtypeStruct((_K, _H), jnp.float32),
                   jax.ShapeDtypeStruct((_K, 8), jnp.float32)],
        scratch_shapes=[pltpu.VMEM((_K, _H), jnp.float32)],
    )(x, m)
    score2, out = pl.pallas_call(
        _out_body,
        grid=(_NBLK,),
        in_specs=[pl.BlockSpec((_BLK, _H), lambda i: (i, 0)),
                  pl.BlockSpec((_K, _H), lambda i: (0, 0)),
                  pl.BlockSpec((_K, _H), lambda i: (0, 0)),
                  pl.BlockSpec((_K, 8), lambda i: (0, 0))],
        out_specs=[pl.BlockSpec((_BLK, _K), lambda i: (i, 0)),
                   pl.BlockSpec((_BLK, _H), lambda i: (i, 0))],
        out_shape=[jax.ShapeDtypeStruct((_N, _K), jnp.float32),
                   jax.ShapeDtypeStruct((_N, _H), jnp.float32)],
        scratch_shapes=[pltpu.VMEM((_K, _H), jnp.float32),
                        pltpu.VMEM((_K, _H), jnp.float32)],
    )(x, m, esumT, cntT)
    return (out, score2)
